# baseline (speedup 1.0000x reference)
import functools
import math

import jax
import jax.numpy as jnp
from jax.experimental import pallas as pl
from jax.experimental.pallas import tpu as pltpu

NUM_EXPERTS = 64
TOP_K = 8
TEMPERATURE = 2.0
LB_WEIGHT = 0.01
CAP_FACTOR = 1.25
BLOCK_TOKENS = 4096


def _gating_kernel(x_ref, g_ref, be_ref, w1_ref, b1_ref, w2_ref, b2_ref,
                   w3_ref, b3_ref,
                   rw_ref, idx_ref, lb_ref, util_ref, cap_ref,
                   gs_acc, ld_acc, *, nblocks, batch, capacity):
    pid = pl.program_id(0)

    @pl.when(pid == 0)
    def _init():
        gs_acc[...] = jnp.zeros_like(gs_acc)
        ld_acc[...] = jnp.zeros_like(ld_acc)

    x = x_ref[...]
    mu = jnp.mean(x, axis=1, keepdims=True)
    xc = x - mu
    var = jnp.mean(xc * xc, axis=1, keepdims=True)
    xn = xc * jax.lax.rsqrt(var + 1e-5) * g_ref[...] + be_ref[...]

    h = jnp.maximum(jnp.dot(xn, w1_ref[...],
                            preferred_element_type=jnp.float32) + b1_ref[...], 0.0)
    h = jnp.maximum(jnp.dot(h, w2_ref[...],
                            preferred_element_type=jnp.float32) + b2_ref[...], 0.0)
    logits = (jnp.dot(h, w3_ref[...],
                      preferred_element_type=jnp.float32) + b3_ref[...]) * (1.0 / TEMPERATURE)

    logits_t = logits.T
    lmax = jnp.max(logits_t, axis=0, keepdims=True)
    e = jnp.exp(logits_t - lmax)
    gates_t = e * (1.0 / jnp.sum(e, axis=0, keepdims=True))

    gs_acc[...] += gates_t

    toks = gates_t.shape[1]
    eidx = jax.lax.broadcasted_iota(
        jnp.int32, (NUM_EXPERTS, toks), 0).astype(jnp.float32)

    m = gates_t
    idx_rows = []
    ck0 = None
    for j in range(TOP_K):
        cm = jnp.max(m, axis=0, keepdims=True)
        if ck0 is None:
            ck0 = cm
        ii = jnp.min(jnp.where(m == cm, eidx, 64.0), axis=0, keepdims=True)
        idx_rows.append(ii)
        m = jnp.where(eidx == ii, -1.0, m)

    idx_ref[...] = jnp.concatenate(idx_rows, axis=0).astype(jnp.int32).T

    sel = m < 0.0
    ew = jnp.exp(gates_t - ck0)
    mew = jnp.where(sel, ew, 0.0)
    rw = mew * (1.0 / jnp.sum(mew, axis=0, keepdims=True))
    rw_ref[...] = rw.T

    ld_acc[...] -= jnp.minimum(m, 0.0)

    @pl.when(pid == nblocks - 1)
    def _finalize():
        gm = jnp.sum(gs_acc[...], axis=1, keepdims=True) * (1.0 / batch)
        entropy = -jnp.sum(gm * jnp.log(gm + 1e-8))
        lb_ref[...] = jnp.full((1, 1), -(1.0 / math.log(NUM_EXPERTS)) * LB_WEIGHT) * entropy
        loads = jnp.sum(ld_acc[...], axis=1, keepdims=True)
        util_ref[...] = loads * (1.0 / batch)
        cap_ref[...] = loads > capacity


def kernel(fingerprint_features, ln_gamma, ln_beta, W1, b1, W2, b2, W3, b3):
    x = fingerprint_features
    batch, fdim = x.shape
    hidden = W1.shape[1]
    inter = W2.shape[1]
    toks = BLOCK_TOKENS if batch % BLOCK_TOKENS == 0 else batch
    nblocks = batch // toks
    capacity = int(batch * CAP_FACTOR / NUM_EXPERTS)

    out_shapes = (
        jax.ShapeDtypeStruct((batch, NUM_EXPERTS), jnp.float32),
        jax.ShapeDtypeStruct((batch, TOP_K), jnp.int32),
        jax.ShapeDtypeStruct((1, 1), jnp.float32),
        jax.ShapeDtypeStruct((NUM_EXPERTS, 1), jnp.float32),
        jax.ShapeDtypeStruct((NUM_EXPERTS, 1), jnp.bool_),
    )
    in_specs = [
        pl.BlockSpec((toks, fdim), lambda i: (i, 0)),
        pl.BlockSpec((1, fdim), lambda i: (0, 0)),
        pl.BlockSpec((1, fdim), lambda i: (0, 0)),
        pl.BlockSpec((fdim, hidden), lambda i: (0, 0)),
        pl.BlockSpec((1, hidden), lambda i: (0, 0)),
        pl.BlockSpec((hidden, inter), lambda i: (0, 0)),
        pl.BlockSpec((1, inter), lambda i: (0, 0)),
        pl.BlockSpec((inter, NUM_EXPERTS), lambda i: (0, 0)),
        pl.BlockSpec((1, NUM_EXPERTS), lambda i: (0, 0)),
    ]
    out_specs = (
        pl.BlockSpec((toks, NUM_EXPERTS), lambda i: (i, 0)),
        pl.BlockSpec((toks, TOP_K), lambda i: (i, 0)),
        pl.BlockSpec((1, 1), lambda i: (0, 0)),
        pl.BlockSpec((NUM_EXPERTS, 1), lambda i: (0, 0)),
        pl.BlockSpec((NUM_EXPERTS, 1), lambda i: (0, 0)),
    )
    rw, idx, lb, util, capf = pl.pallas_call(
        functools.partial(_gating_kernel, nblocks=nblocks, batch=batch,
                          capacity=capacity),
        grid=(nblocks,),
        in_specs=in_specs,
        out_specs=out_specs,
        out_shape=out_shapes,
        scratch_shapes=[
            pltpu.VMEM((NUM_EXPERTS, toks), jnp.float32),
            pltpu.VMEM((NUM_EXPERTS, toks), jnp.float32),
        ],
    )(x, ln_gamma.reshape(1, fdim), ln_beta.reshape(1, fdim),
      W1, b1.reshape(1, hidden), W2, b2.reshape(1, inter),
      W3, b3.reshape(1, NUM_EXPERTS))

    return (rw, idx, lb.reshape(()), util.reshape(NUM_EXPERTS),
            capf.reshape(NUM_EXPERTS))


# speedup vs baseline: 1.0203x; 1.0203x over previous
import functools
import math

import jax
import jax.numpy as jnp
from jax.experimental import pallas as pl
from jax.experimental.pallas import tpu as pltpu

NUM_EXPERTS = 64
TOP_K = 8
TEMPERATURE = 2.0
LB_WEIGHT = 0.01
CAP_FACTOR = 1.25
BLOCK_TOKENS = 2048


def _gating_kernel(x_ref, g_ref, be_ref, w1_ref, b1_ref, w2_ref, b2_ref,
                   w3_ref, b3_ref,
                   rw_ref, idx_ref, lb_ref, util_ref, cap_ref,
                   gs_acc, ld_acc, *, nblocks, batch, capacity):
    pid = pl.program_id(0)

    @pl.when(pid == 0)
    def _init():
        gs_acc[...] = jnp.zeros_like(gs_acc)
        ld_acc[...] = jnp.zeros_like(ld_acc)

    x = x_ref[...]
    mu = jnp.mean(x, axis=1, keepdims=True)
    xc = x - mu
    var = jnp.mean(xc * xc, axis=1, keepdims=True)
    xn = xc * jax.lax.rsqrt(var + 1e-5) * g_ref[...] + be_ref[...]

    h = jnp.maximum(jnp.dot(xn, w1_ref[...],
                            preferred_element_type=jnp.float32) + b1_ref[...], 0.0)
    h = jnp.maximum(jnp.dot(h, w2_ref[...],
                            preferred_element_type=jnp.float32) + b2_ref[...], 0.0)
    logits = (jnp.dot(h, w3_ref[...],
                      preferred_element_type=jnp.float32) + b3_ref[...]) * (1.0 / TEMPERATURE)

    logits_t = logits.T
    lmax = jnp.max(logits_t, axis=0, keepdims=True)
    e = jnp.exp(logits_t - lmax)
    inv_d = 1.0 / jnp.sum(e, axis=0, keepdims=True)
    gates_t = e * inv_d

    gs_acc[...] += gates_t

    toks = gates_t.shape[1]
    eidx = jax.lax.broadcasted_iota(
        jnp.int32, (NUM_EXPERTS, toks), 0).astype(jnp.float32)

    m = gates_t
    idx_rows = []
    ck0 = None
    for j in range(TOP_K):
        cm = inv_d if j == 0 else jnp.max(m, axis=0, keepdims=True)
        if ck0 is None:
            ck0 = cm
        ii = jnp.min(jnp.where(m == cm, eidx, 64.0), axis=0, keepdims=True)
        idx_rows.append(ii)
        m = jnp.where(eidx == ii, -1.0, m)

    idx_ref[...] = jnp.concatenate(idx_rows, axis=0).astype(jnp.int32).T

    sel = m < 0.0
    ew = jnp.exp(gates_t - ck0)
    mew = jnp.where(sel, ew, 0.0)
    rw = mew * (1.0 / jnp.sum(mew, axis=0, keepdims=True))
    rw_ref[...] = rw.T

    ld_acc[...] -= jnp.minimum(m, 0.0)

    @pl.when(pid == nblocks - 1)
    def _finalize():
        gm = jnp.sum(gs_acc[...], axis=1, keepdims=True) * (1.0 / batch)
        entropy = -jnp.sum(gm * jnp.log(gm + 1e-8))
        lb_ref[...] = jnp.full((1, 1), -(1.0 / math.log(NUM_EXPERTS)) * LB_WEIGHT) * entropy
        loads = jnp.sum(ld_acc[...], axis=1, keepdims=True)
        util_ref[...] = loads * (1.0 / batch)
        cap_ref[...] = loads > capacity


def kernel(fingerprint_features, ln_gamma, ln_beta, W1, b1, W2, b2, W3, b3):
    x = fingerprint_features
    batch, fdim = x.shape
    hidden = W1.shape[1]
    inter = W2.shape[1]
    toks = BLOCK_TOKENS if batch % BLOCK_TOKENS == 0 else batch
    nblocks = batch // toks
    capacity = int(batch * CAP_FACTOR / NUM_EXPERTS)

    out_shapes = (
        jax.ShapeDtypeStruct((batch, NUM_EXPERTS), jnp.float32),
        jax.ShapeDtypeStruct((batch, TOP_K), jnp.int32),
        jax.ShapeDtypeStruct((1, 1), jnp.float32),
        jax.ShapeDtypeStruct((NUM_EXPERTS, 1), jnp.float32),
        jax.ShapeDtypeStruct((NUM_EXPERTS, 1), jnp.bool_),
    )
    in_specs = [
        pl.BlockSpec((toks, fdim), lambda i: (i, 0)),
        pl.BlockSpec((1, fdim), lambda i: (0, 0)),
        pl.BlockSpec((1, fdim), lambda i: (0, 0)),
        pl.BlockSpec((fdim, hidden), lambda i: (0, 0)),
        pl.BlockSpec((1, hidden), lambda i: (0, 0)),
        pl.BlockSpec((hidden, inter), lambda i: (0, 0)),
        pl.BlockSpec((1, inter), lambda i: (0, 0)),
        pl.BlockSpec((inter, NUM_EXPERTS), lambda i: (0, 0)),
        pl.BlockSpec((1, NUM_EXPERTS), lambda i: (0, 0)),
    ]
    out_specs = (
        pl.BlockSpec((toks, NUM_EXPERTS), lambda i: (i, 0)),
        pl.BlockSpec((toks, TOP_K), lambda i: (i, 0)),
        pl.BlockSpec((1, 1), lambda i: (0, 0)),
        pl.BlockSpec((NUM_EXPERTS, 1), lambda i: (0, 0)),
        pl.BlockSpec((NUM_EXPERTS, 1), lambda i: (0, 0)),
    )
    rw, idx, lb, util, capf = pl.pallas_call(
        functools.partial(_gating_kernel, nblocks=nblocks, batch=batch,
                          capacity=capacity),
        grid=(nblocks,),
        in_specs=in_specs,
        out_specs=out_specs,
        out_shape=out_shapes,
        scratch_shapes=[
            pltpu.VMEM((NUM_EXPERTS, toks), jnp.float32),
            pltpu.VMEM((NUM_EXPERTS, toks), jnp.float32),
        ],
    )(x, ln_gamma.reshape(1, fdim), ln_beta.reshape(1, fdim),
      W1, b1.reshape(1, hidden), W2, b2.reshape(1, inter),
      W3, b3.reshape(1, NUM_EXPERTS))

    return (rw, idx, lb.reshape(()), util.reshape(NUM_EXPERTS),
            capf.reshape(NUM_EXPERTS))
